# baseline (speedup 1.0000x reference)
import functools

import jax
import jax.numpy as jnp
from jax import lax
from jax.experimental import pallas as pl
from jax.experimental.pallas import tpu as pltpu
from jax.experimental.pallas import tpu_sc as plsc

F32 = jnp.float32
BF16 = jnp.bfloat16

N = 10000
E = 160000
M = 160000
FD = 128


NC, NS = 2, 16
NW = NC * NS
CH = 128
NCHUNK = M // CH
ITERS = (NCHUNK + NW - 1) // NW


def _sc_gather(nidx, eidx, feat, hu):
    mesh = plsc.VectorSubcoreMesh(core_axis_name="c", subcore_axis_name="s")

    @functools.partial(
        pl.kernel,
        mesh=mesh,
        out_type=(
            jax.ShapeDtypeStruct((M, 128), F32),
            jax.ShapeDtypeStruct((M, 128), jnp.int32),
        ),
        scratch_types=[
            pltpu.VMEM((CH,), jnp.int32),
            pltpu.VMEM((CH,), jnp.int32),
            pltpu.VMEM((CH, 128), F32),
            pltpu.VMEM((CH, 128), jnp.int32),
            pltpu.VMEM((CH,), jnp.int32),
            pltpu.VMEM((CH,), jnp.int32),
            pltpu.VMEM((CH, 128), F32),
            pltpu.VMEM((CH, 128), jnp.int32),
            pltpu.SemaphoreType.DMA,
            pltpu.SemaphoreType.DMA,
            pltpu.SemaphoreType.DMA,
            pltpu.SemaphoreType.DMA,
            pltpu.SemaphoreType.DMA,
            pltpu.SemaphoreType.DMA,
            pltpu.SemaphoreType.DMA,
            pltpu.SemaphoreType.DMA,
        ],
    )
    def gather_k(nidx_hbm, eidx_hbm, feat_hbm, hu_hbm,
                 fe_out, hu_out,
                 nv0, ev0, fv0, uv0, nv1, ev1, fv1, uv1,
                 sf0, sh0, wf0, wh0, sf1, sh1, wf1, wh1):
        wid = lax.axis_index("s") * NC + lax.axis_index("c")
        bufs = [(nv0, ev0, fv0, uv0, sf0, sh0, wf0, wh0),
                (nv1, ev1, fv1, uv1, sf1, sh1, wf1, wh1)]

        def load_and_fire(k, b):
            nv, ev, fv, uv, sf, sh, _, _ = bufs[b]
            c = wid + k * NW

            @pl.when(c < NCHUNK)
            def _():
                off = c * CH
                pltpu.sync_copy(nidx_hbm.at[pl.ds(off, CH)], nv)
                pltpu.sync_copy(eidx_hbm.at[pl.ds(off, CH)], ev)
                pltpu.async_copy(feat_hbm.at[ev], fv, sf)
                pltpu.async_copy(hu_hbm.at[nv], uv, sh)

        def wait_wb(k, b):
            _, _, fv, uv, _, _, wf, wh = bufs[b]
            c = wid + k * NW

            @pl.when(jnp.logical_and(c >= 0, c < NCHUNK))
            def _():
                off = c * CH
                pltpu.make_async_copy(
                    fv, fe_out.at[pl.ds(off, CH)], wf).wait()
                pltpu.make_async_copy(
                    uv, hu_out.at[pl.ds(off, CH)], wh).wait()

        def drain_and_store(k, b):
            nv, ev, fv, uv, sf, sh, wf, wh = bufs[b]
            c = wid + k * NW

            @pl.when(c < NCHUNK)
            def _():
                off = c * CH
                pltpu.make_async_copy(feat_hbm.at[ev], fv, sf).wait()
                pltpu.make_async_copy(hu_hbm.at[nv], uv, sh).wait()
                pltpu.async_copy(fv, fe_out.at[pl.ds(off, CH)], wf)
                pltpu.async_copy(uv, hu_out.at[pl.ds(off, CH)], wh)

        load_and_fire(0, 0)

        def outer(k2, carry):
            for b in (0, 1):
                k = k2 * 2 + b
                wait_wb(k - 1, 1 - b)
                load_and_fire(k + 1, 1 - b)
                drain_and_store(k, b)
            return carry

        lax.fori_loop(0, (ITERS + 1) // 2, outer, 0)

        wait_wb(ITERS - 1, (ITERS - 1) % 2)

    return gather_k(nidx, eidx, feat, hu)


MB = 2000
NBLK = M // MB


def _mlp_body(fe_ref, hu_ref,
              wpt_ref, bphi_ref,
              w1a_ref, w1b_ref, w1c_ref, b1_ref,
              w2_ref, b2_ref, w3_ref, b3_ref,
              wv_ref, bv_ref,
              av_out, val_out, acc_ref):
    i = pl.program_id(0)

    @pl.when(i == 0)
    def _():
        acc_ref[...] = jnp.zeros_like(acc_ref)

    ae = (jnp.dot(fe_ref[...].astype(BF16), wpt_ref[...],
                  preferred_element_type=F32)
          + bphi_ref[...])
    hu32 = hu_ref[...]
    lo = jax.lax.bitcast_convert_type(hu32 << 16, F32)
    hi = jax.lax.bitcast_convert_type(
        hu32 & jnp.int32(-65536), F32)
    hg = lo[:, 0:64]
    ug = jnp.concatenate([lo[:, 64:128], hi[:, 0:64]], axis=1)
    re = jnp.maximum(ae, 0.0)
    ah = ae * hg
    n2 = jnp.sum(ah * ah, axis=1, keepdims=True)
    ah = ah / jnp.maximum(jnp.sqrt(n2), 1e-12)

    acc_ref[...] += jnp.concatenate(
        [re.sum(axis=0, keepdims=True),
         ah.sum(axis=0, keepdims=True),
         ug.sum(axis=0, keepdims=True)], axis=1)

    x = (jnp.dot(re.astype(BF16), w1a_ref[...], preferred_element_type=F32)
         + jnp.dot(ah.astype(BF16), w1b_ref[...], preferred_element_type=F32)
         + jnp.dot(ug.astype(BF16), w1c_ref[...], preferred_element_type=F32)
         + b1_ref[...])
    x = jnp.maximum(x, 0.0)
    x = jnp.maximum(
        jnp.dot(x.astype(BF16), w2_ref[...], preferred_element_type=F32)
        + b2_ref[...], 0.0)
    av_out[...] = (jnp.dot(x.astype(BF16), w3_ref[...],
                           preferred_element_type=F32)
                   + b3_ref[...])

    @pl.when(i == NBLK - 1)
    def _():
        mean = acc_ref[...] * (1.0 / M)
        val_out[...] = (
            jnp.dot(mean, wv_ref[...], preferred_element_type=F32)
            + bv_ref[...])


def _mlp(fe, hu, wpt, bphi_row, w1a, w1b, w1c, b1_row, w2t, b2_row, w3t,
         b3_11, wvt, bv_11):
    return pl.pallas_call(
        _mlp_body,
        grid=(NBLK,),
        in_specs=[
            pl.BlockSpec((MB, 128), lambda i: (i, 0)),
            pl.BlockSpec((MB, 128), lambda i: (i, 0)),
            pl.BlockSpec((128, 64), lambda i: (0, 0)),
            pl.BlockSpec((1, 64), lambda i: (0, 0)),
            pl.BlockSpec((64, 64), lambda i: (0, 0)),
            pl.BlockSpec((64, 64), lambda i: (0, 0)),
            pl.BlockSpec((128, 64), lambda i: (0, 0)),
            pl.BlockSpec((1, 64), lambda i: (0, 0)),
            pl.BlockSpec((64, 64), lambda i: (0, 0)),
            pl.BlockSpec((1, 64), lambda i: (0, 0)),
            pl.BlockSpec((64, 1), lambda i: (0, 0)),
            pl.BlockSpec((1, 1), lambda i: (0, 0)),
            pl.BlockSpec((256, 1), lambda i: (0, 0)),
            pl.BlockSpec((1, 1), lambda i: (0, 0)),
        ],
        out_specs=[
            pl.BlockSpec((MB, 1), lambda i: (i, 0)),
            pl.BlockSpec((1, 1), lambda i: (0, 0)),
        ],
        out_shape=[
            jax.ShapeDtypeStruct((M, 1), F32),
            jax.ShapeDtypeStruct((1, 1), F32),
        ],
        scratch_shapes=[pltpu.VMEM((1, 256), F32)],
    )(fe, hu, wpt, bphi_row, w1a, w1b, w1c, b1_row, w2t, b2_row, w3t,
      b3_11, wvt, bv_11)


def kernel(h, cand_edge_feat, u_1, c_adj, j,
           W_phi, b_phi, W1, b1, W2, b2, W3, b3, Wv, bv):
    nidx = c_adj[0]
    eidx = c_adj[1] - j

    lo_bits = jax.lax.bitcast_convert_type(
        jnp.concatenate([h, u_1[:, 0:64]], axis=1).astype(BF16),
        jnp.uint16).astype(jnp.uint32)
    hi_bits = jax.lax.bitcast_convert_type(
        jnp.concatenate([u_1[:, 64:128], jnp.zeros((N, 64), F32)], axis=1)
        .astype(BF16), jnp.uint16).astype(jnp.uint32)
    hu = jax.lax.bitcast_convert_type(
        lo_bits | (hi_bits << 16), jnp.int32)

    fe, hug = _sc_gather(nidx, eidx, cand_edge_feat, hu)

    w1t = W1.T.astype(BF16)
    a_values, value = _mlp(
        fe, hug, W_phi.T.astype(BF16), b_phi.reshape(1, 64),
        w1t[0:64], w1t[64:128], w1t[128:256], b1.reshape(1, 64),
        W2.T.astype(BF16), b2.reshape(1, 64), W3.T.astype(BF16),
        b3.reshape(1, 1), Wv.T, bv.reshape(1, 1))

    return (value, a_values.reshape(1, M))


# speedup vs baseline: 1.0843x; 1.0843x over previous
import functools

import jax
import jax.numpy as jnp
from jax import lax
from jax.experimental import pallas as pl
from jax.experimental.pallas import tpu as pltpu
from jax.experimental.pallas import tpu_sc as plsc

F32 = jnp.float32
BF16 = jnp.bfloat16

N = 10000
E = 160000
M = 160000
FD = 128

NSPLIT = 5
MC = M // NSPLIT


NC, NS = 2, 16
NW = NC * NS
CH = 128
NCHUNK = MC // CH
ITERS = (NCHUNK + NW - 1) // NW


def _sc_gather(nidx, eidx, feat, hu):
    mesh = plsc.VectorSubcoreMesh(core_axis_name="c", subcore_axis_name="s")

    @functools.partial(
        pl.kernel,
        mesh=mesh,
        out_type=(
            jax.ShapeDtypeStruct((MC, 128), F32),
            jax.ShapeDtypeStruct((MC, 128), jnp.int32),
        ),
        scratch_types=[
            pltpu.VMEM((CH,), jnp.int32),
            pltpu.VMEM((CH,), jnp.int32),
            pltpu.VMEM((CH, 128), F32),
            pltpu.VMEM((CH, 128), jnp.int32),
            pltpu.VMEM((CH,), jnp.int32),
            pltpu.VMEM((CH,), jnp.int32),
            pltpu.VMEM((CH, 128), F32),
            pltpu.VMEM((CH, 128), jnp.int32),
            pltpu.SemaphoreType.DMA,
            pltpu.SemaphoreType.DMA,
            pltpu.SemaphoreType.DMA,
            pltpu.SemaphoreType.DMA,
            pltpu.SemaphoreType.DMA,
            pltpu.SemaphoreType.DMA,
            pltpu.SemaphoreType.DMA,
            pltpu.SemaphoreType.DMA,
        ],
    )
    def gather_k(nidx_hbm, eidx_hbm, feat_hbm, hu_hbm,
                 fe_out, hu_out,
                 nv0, ev0, fv0, uv0, nv1, ev1, fv1, uv1,
                 sf0, sh0, wf0, wh0, sf1, sh1, wf1, wh1):
        wid = lax.axis_index("s") * NC + lax.axis_index("c")
        bufs = [(nv0, ev0, fv0, uv0, sf0, sh0, wf0, wh0),
                (nv1, ev1, fv1, uv1, sf1, sh1, wf1, wh1)]

        def load_and_fire(k, b):
            nv, ev, fv, uv, sf, sh, _, _ = bufs[b]
            c = wid + k * NW

            @pl.when(c < NCHUNK)
            def _():
                off = c * CH
                pltpu.sync_copy(nidx_hbm.at[pl.ds(off, CH)], nv)
                pltpu.sync_copy(eidx_hbm.at[pl.ds(off, CH)], ev)
                pltpu.async_copy(feat_hbm.at[ev], fv, sf)
                pltpu.async_copy(hu_hbm.at[nv], uv, sh)

        def wait_wb(k, b):
            _, _, fv, uv, _, _, wf, wh = bufs[b]
            c = wid + k * NW

            @pl.when(jnp.logical_and(c >= 0, c < NCHUNK))
            def _():
                off = c * CH
                pltpu.make_async_copy(
                    fv, fe_out.at[pl.ds(off, CH)], wf).wait()
                pltpu.make_async_copy(
                    uv, hu_out.at[pl.ds(off, CH)], wh).wait()

        def drain_and_store(k, b):
            nv, ev, fv, uv, sf, sh, wf, wh = bufs[b]
            c = wid + k * NW

            @pl.when(c < NCHUNK)
            def _():
                off = c * CH
                pltpu.make_async_copy(feat_hbm.at[ev], fv, sf).wait()
                pltpu.make_async_copy(hu_hbm.at[nv], uv, sh).wait()
                pltpu.async_copy(fv, fe_out.at[pl.ds(off, CH)], wf)
                pltpu.async_copy(uv, hu_out.at[pl.ds(off, CH)], wh)

        load_and_fire(0, 0)

        def outer(k2, carry):
            for b in (0, 1):
                k = k2 * 2 + b
                wait_wb(k - 1, 1 - b)
                load_and_fire(k + 1, 1 - b)
                drain_and_store(k, b)
            return carry

        lax.fori_loop(0, (ITERS + 1) // 2, outer, 0)

        wait_wb(ITERS - 1, (ITERS - 1) % 2)

    return gather_k(nidx, eidx, feat, hu)


MB = 2000
NBLK = MC // MB


def _mlp_body(fe_ref, hu_ref,
              wpt_ref, bphi_ref,
              w1a_ref, w1b_ref, w1c_ref, b1_ref,
              w2_ref, b2_ref, w3_ref, b3_ref,
              av_out, psum_out, acc_ref):
    i = pl.program_id(0)

    @pl.when(i == 0)
    def _():
        acc_ref[...] = jnp.zeros_like(acc_ref)

    ae = (jnp.dot(fe_ref[...].astype(BF16), wpt_ref[...],
                  preferred_element_type=F32)
          + bphi_ref[...])
    hu32 = hu_ref[...]
    ug = jax.lax.bitcast_convert_type(hu32 << 16, F32)
    hg = jax.lax.bitcast_convert_type(
        hu32 & jnp.int32(-65536), F32)[:, 0:64]
    re = jnp.maximum(ae, 0.0)
    ah = ae * hg
    n2 = jnp.sum(ah * ah, axis=1, keepdims=True)
    ah = ah / jnp.maximum(jnp.sqrt(n2), 1e-12)

    acc_ref[...] += jnp.concatenate(
        [re.sum(axis=0, keepdims=True),
         ah.sum(axis=0, keepdims=True),
         ug.sum(axis=0, keepdims=True)], axis=1)

    x = (jnp.dot(re.astype(BF16), w1a_ref[...], preferred_element_type=F32)
         + jnp.dot(ah.astype(BF16), w1b_ref[...], preferred_element_type=F32)
         + jnp.dot(ug.astype(BF16), w1c_ref[...], preferred_element_type=F32)
         + b1_ref[...])
    x = jnp.maximum(x, 0.0)
    x = jnp.maximum(
        jnp.dot(x.astype(BF16), w2_ref[...], preferred_element_type=F32)
        + b2_ref[...], 0.0)
    av_out[...] = (jnp.dot(x.astype(BF16), w3_ref[...],
                           preferred_element_type=F32)
                   + b3_ref[...])

    @pl.when(i == NBLK - 1)
    def _():
        psum_out[...] = acc_ref[...]


def _mlp(fe, hu, wpt, bphi_row, w1a, w1b, w1c, b1_row, w2t, b2_row, w3t,
         b3_11):
    return pl.pallas_call(
        _mlp_body,
        grid=(NBLK,),
        in_specs=[
            pl.BlockSpec((MB, 128), lambda i: (i, 0)),
            pl.BlockSpec((MB, 128), lambda i: (i, 0)),
            pl.BlockSpec((128, 64), lambda i: (0, 0)),
            pl.BlockSpec((1, 64), lambda i: (0, 0)),
            pl.BlockSpec((64, 64), lambda i: (0, 0)),
            pl.BlockSpec((64, 64), lambda i: (0, 0)),
            pl.BlockSpec((128, 64), lambda i: (0, 0)),
            pl.BlockSpec((1, 64), lambda i: (0, 0)),
            pl.BlockSpec((64, 64), lambda i: (0, 0)),
            pl.BlockSpec((1, 64), lambda i: (0, 0)),
            pl.BlockSpec((64, 1), lambda i: (0, 0)),
            pl.BlockSpec((1, 1), lambda i: (0, 0)),
        ],
        out_specs=[
            pl.BlockSpec((MB, 1), lambda i: (i, 0)),
            pl.BlockSpec((1, 256), lambda i: (0, 0)),
        ],
        out_shape=[
            jax.ShapeDtypeStruct((MC, 1), F32),
            jax.ShapeDtypeStruct((1, 256), F32),
        ],
        scratch_shapes=[pltpu.VMEM((1, 256), F32)],
    )(fe, hu, wpt, bphi_row, w1a, w1b, w1c, b1_row, w2t, b2_row, w3t, b3_11)


def _value_body(ps_ref, wv_ref, bv_ref, val_out):
    mean = jnp.sum(ps_ref[...], axis=0, keepdims=True) * (1.0 / M)
    val_out[...] = (jnp.dot(mean.astype(BF16), wv_ref[...],
                            preferred_element_type=F32)
                    + bv_ref[...])


def _value(psums, wvt, bv_11):
    return pl.pallas_call(
        _value_body,
        in_specs=[
            pl.BlockSpec((NSPLIT, 256), lambda: (0, 0)),
            pl.BlockSpec((256, 1), lambda: (0, 0)),
            pl.BlockSpec((1, 1), lambda: (0, 0)),
        ],
        out_specs=pl.BlockSpec((1, 1), lambda: (0, 0)),
        out_shape=jax.ShapeDtypeStruct((1, 1), F32),
    )(psums, wvt, bv_11)


def kernel(h, cand_edge_feat, u_1, c_adj, j,
           W_phi, b_phi, W1, b1, W2, b2, W3, b3, Wv, bv):
    nidx = c_adj[0]
    eidx = c_adj[1] - j

    lo_bits = jax.lax.bitcast_convert_type(
        u_1.astype(BF16), jnp.uint16).astype(jnp.uint32)
    hi_bits = jax.lax.bitcast_convert_type(
        jnp.concatenate([h, jnp.zeros((N, 64), F32)], axis=1)
        .astype(BF16), jnp.uint16).astype(jnp.uint32)
    hu = jax.lax.bitcast_convert_type(
        lo_bits | (hi_bits << 16), jnp.int32)

    w1t = W1.T.astype(BF16)
    wpt = W_phi.T.astype(BF16)
    w2t = W2.T.astype(BF16)
    w3t = W3.T.astype(BF16)

    av_parts = []
    ps_parts = []
    for s in range(NSPLIT):
        fe, hug = _sc_gather(
            lax.slice(nidx, (s * MC,), ((s + 1) * MC,)),
            lax.slice(eidx, (s * MC,), ((s + 1) * MC,)),
            cand_edge_feat, hu)
        av, ps = _mlp(
            fe, hug, wpt, b_phi.reshape(1, 64),
            w1t[0:64], w1t[64:128], w1t[128:256], b1.reshape(1, 64),
            w2t, b2.reshape(1, 64), w3t, b3.reshape(1, 1))
        av_parts.append(av)
        ps_parts.append(ps)

    value = _value(jnp.concatenate(ps_parts, axis=0), Wv.T.astype(BF16),
                   bv.reshape(1, 1))
    a_values = jnp.concatenate(av_parts, axis=0).reshape(1, M)

    return (value, a_values)


# speedup vs baseline: 1.1087x; 1.0225x over previous
import functools

import jax
import jax.numpy as jnp
from jax import lax
from jax.experimental import pallas as pl
from jax.experimental.pallas import tpu as pltpu
from jax.experimental.pallas import tpu_sc as plsc

F32 = jnp.float32
BF16 = jnp.bfloat16

N = 10000
E = 160000
M = 160000
FD = 128

NSPLIT = 5
MC = M // NSPLIT


NC, NS = 2, 16
NW = NC * NS
CH = 128
NCHUNK = MC // CH
ITERS = (NCHUNK + NW - 1) // NW
NP = 10112


def _sc_gather(nidx, eidx, feat, hu):
    mesh = plsc.VectorSubcoreMesh(core_axis_name="c", subcore_axis_name="s")

    @functools.partial(
        pl.kernel,
        mesh=mesh,
        out_type=(
            jax.ShapeDtypeStruct((MC, 128), F32),
            jax.ShapeDtypeStruct((MC, 128), jnp.int32),
        ),
        scratch_types=[
            pltpu.VMEM((CH,), jnp.int32),
            pltpu.VMEM((CH,), jnp.int32),
            pltpu.VMEM((CH, 128), F32),
            pltpu.VMEM((CH, 128), jnp.int32),
            pltpu.VMEM((CH,), jnp.int32),
            pltpu.VMEM((CH,), jnp.int32),
            pltpu.VMEM((CH, 128), F32),
            pltpu.VMEM((CH, 128), jnp.int32),
            pltpu.SemaphoreType.DMA,
            pltpu.SemaphoreType.DMA,
            pltpu.SemaphoreType.DMA,
            pltpu.SemaphoreType.DMA,
            pltpu.SemaphoreType.DMA,
            pltpu.SemaphoreType.DMA,
            pltpu.SemaphoreType.DMA,
            pltpu.SemaphoreType.DMA,
        ],
    )
    def gather_k(nidx_hbm, eidx_hbm, feat_hbm, hu_hbm,
                 fe_out, hu_out,
                 nv0, ev0, fv0, uv0, nv1, ev1, fv1, uv1,
                 sf0, sh0, wf0, wh0, sf1, sh1, wf1, wh1):
        wid = lax.axis_index("s") * NC + lax.axis_index("c")
        bufs = [(nv0, ev0, fv0, uv0, sf0, sh0, wf0, wh0),
                (nv1, ev1, fv1, uv1, sf1, sh1, wf1, wh1)]

        def load_and_fire(k, b):
            nv, ev, fv, uv, sf, sh, _, _ = bufs[b]
            c = wid + k * NW

            @pl.when(c < NCHUNK)
            def _():
                off = c * CH
                pltpu.sync_copy(nidx_hbm.at[pl.ds(off, CH)], nv)
                pltpu.sync_copy(eidx_hbm.at[pl.ds(off, CH)], ev)
                pltpu.async_copy(feat_hbm.at[ev], fv, sf)
                pltpu.async_copy(hu_hbm.at[nv], uv, sh)

        def wait_wb(k, b):
            _, _, fv, uv, _, _, wf, wh = bufs[b]
            c = wid + k * NW

            @pl.when(jnp.logical_and(c >= 0, c < NCHUNK))
            def _():
                off = c * CH
                pltpu.make_async_copy(
                    fv, fe_out.at[pl.ds(off, CH)], wf).wait()
                pltpu.make_async_copy(
                    uv, hu_out.at[pl.ds(off, CH)], wh).wait()

        def drain_and_store(k, b):
            nv, ev, fv, uv, sf, sh, wf, wh = bufs[b]
            c = wid + k * NW

            @pl.when(c < NCHUNK)
            def _():
                off = c * CH
                pltpu.make_async_copy(feat_hbm.at[ev], fv, sf).wait()
                pltpu.make_async_copy(hu_hbm.at[nv], uv, sh).wait()
                pltpu.async_copy(fv, fe_out.at[pl.ds(off, CH)], wf)
                pltpu.async_copy(uv, hu_out.at[pl.ds(off, CH)], wh)

        load_and_fire(0, 0)

        def outer(k2, carry):
            for b in (0, 1):
                k = k2 * 2 + b
                wait_wb(k - 1, 1 - b)
                load_and_fire(k + 1, 1 - b)
                drain_and_store(k, b)
            return carry

        lax.fori_loop(0, (ITERS + 1) // 2, outer, 0)

        wait_wb(ITERS - 1, (ITERS - 1) % 2)

    return gather_k(nidx, eidx, feat, hu)


MB = 2000
NBLK = MC // MB


def _mlp_body(fe_ref, hu_ref,
              wpt_ref, bphi_ref,
              w1a_ref, w1b_ref, w1cl_ref, w1ch_ref, b1_ref,
              w2_ref, b2_ref, w3_ref, b3_ref,
              av_out, psum_out, acc_ref):
    i = pl.program_id(0)

    @pl.when(i == 0)
    def _():
        acc_ref[...] = jnp.zeros_like(acc_ref)

    ae = (jnp.dot(fe_ref[...], wpt_ref[...], preferred_element_type=F32)
          + bphi_ref[...])
    hu32 = hu_ref[...]
    hg = jax.lax.bitcast_convert_type(hu32, F32)[:, 0:64]
    up = hu32[:, 64:128]
    ul = jax.lax.bitcast_convert_type(up << 16, F32)
    uh = jax.lax.bitcast_convert_type(up & jnp.int32(-65536), F32)
    re = jnp.maximum(ae, 0.0)
    ah = ae * hg
    n2 = jnp.sum(ah * ah, axis=1, keepdims=True)
    ah = ah / jnp.maximum(jnp.sqrt(n2), 1e-12)

    acc_ref[...] += jnp.concatenate(
        [re.sum(axis=0, keepdims=True),
         ah.sum(axis=0, keepdims=True),
         ul.sum(axis=0, keepdims=True),
         uh.sum(axis=0, keepdims=True)], axis=1)

    x = (jnp.dot(re, w1a_ref[...], preferred_element_type=F32)
         + jnp.dot(ah, w1b_ref[...], preferred_element_type=F32)
         + jnp.dot(ul, w1cl_ref[...], preferred_element_type=F32)
         + jnp.dot(uh, w1ch_ref[...], preferred_element_type=F32)
         + b1_ref[...])
    x = jnp.maximum(x, 0.0)
    x = jnp.maximum(
        jnp.dot(x, w2_ref[...], preferred_element_type=F32) + b2_ref[...], 0.0)
    av_out[...] = (jnp.dot(x, w3_ref[...], preferred_element_type=F32)
                   + b3_ref[...])

    @pl.when(i == NBLK - 1)
    def _():
        psum_out[...] = acc_ref[...]


def _mlp(fe, hu, wpt, bphi_row, w1a, w1b, w1cl, w1ch, b1_row, w2t, b2_row,
         w3t, b3_11):
    return pl.pallas_call(
        _mlp_body,
        grid=(NBLK,),
        in_specs=[
            pl.BlockSpec((MB, 128), lambda i: (i, 0)),
            pl.BlockSpec((MB, 128), lambda i: (i, 0)),
            pl.BlockSpec((128, 64), lambda i: (0, 0)),
            pl.BlockSpec((1, 64), lambda i: (0, 0)),
            pl.BlockSpec((64, 64), lambda i: (0, 0)),
            pl.BlockSpec((64, 64), lambda i: (0, 0)),
            pl.BlockSpec((64, 64), lambda i: (0, 0)),
            pl.BlockSpec((64, 64), lambda i: (0, 0)),
            pl.BlockSpec((1, 64), lambda i: (0, 0)),
            pl.BlockSpec((64, 64), lambda i: (0, 0)),
            pl.BlockSpec((1, 64), lambda i: (0, 0)),
            pl.BlockSpec((64, 1), lambda i: (0, 0)),
            pl.BlockSpec((1, 1), lambda i: (0, 0)),
        ],
        out_specs=[
            pl.BlockSpec((MB, 1), lambda i: (i, 0)),
            pl.BlockSpec((1, 256), lambda i: (0, 0)),
        ],
        out_shape=[
            jax.ShapeDtypeStruct((MC, 1), F32),
            jax.ShapeDtypeStruct((1, 256), F32),
        ],
        scratch_shapes=[pltpu.VMEM((1, 256), F32)],
    )(fe, hu, wpt, bphi_row, w1a, w1b, w1cl, w1ch, b1_row, w2t, b2_row, w3t,
      b3_11)


def _value_body(ps_ref, wv_ref, bv_ref, val_out):
    mean = jnp.sum(ps_ref[...], axis=0, keepdims=True) * (1.0 / M)
    val_out[...] = (jnp.dot(mean, wv_ref[...], preferred_element_type=F32)
                    + bv_ref[...])


def _value(psums, wvt, bv_11):
    return pl.pallas_call(
        _value_body,
        in_specs=[
            pl.BlockSpec((NSPLIT, 256), lambda: (0, 0)),
            pl.BlockSpec((256, 1), lambda: (0, 0)),
            pl.BlockSpec((1, 1), lambda: (0, 0)),
        ],
        out_specs=pl.BlockSpec((1, 1), lambda: (0, 0)),
        out_shape=jax.ShapeDtypeStruct((1, 1), F32),
    )(psums, wvt, bv_11)


def kernel(h, cand_edge_feat, u_1, c_adj, j,
           W_phi, b_phi, W1, b1, W2, b2, W3, b3, Wv, bv):
    nidx = c_adj[0]
    eidx = c_adj[1] - j

    h_bits = jax.lax.bitcast_convert_type(h, jnp.int32)
    lo_bits = jax.lax.bitcast_convert_type(
        u_1[:, 0:64].astype(BF16), jnp.uint16).astype(jnp.uint32)
    hi_bits = jax.lax.bitcast_convert_type(
        u_1[:, 64:128].astype(BF16), jnp.uint16).astype(jnp.uint32)
    u_pack = jax.lax.bitcast_convert_type(
        lo_bits | (hi_bits << 16), jnp.int32)
    hu = jnp.concatenate([h_bits, u_pack], axis=1)

    w1t = W1.T
    wpt = W_phi.T
    w2t = W2.T
    w3t = W3.T

    av_parts = []
    ps_parts = []
    for s in range(NSPLIT):
        fe, hug = _sc_gather(
            lax.slice(nidx, (s * MC,), ((s + 1) * MC,)),
            lax.slice(eidx, (s * MC,), ((s + 1) * MC,)),
            cand_edge_feat, hu)
        av, ps = _mlp(
            fe, hug, wpt, b_phi.reshape(1, 64),
            w1t[0:64], w1t[64:128], w1t[128:192], w1t[192:256],
            b1.reshape(1, 64),
            w2t, b2.reshape(1, 64), w3t, b3.reshape(1, 1))
        av_parts.append(av)
        ps_parts.append(ps)

    value = _value(jnp.concatenate(ps_parts, axis=0), Wv.T,
                   bv.reshape(1, 1))
    a_values = jnp.concatenate(av_parts, axis=0).reshape(1, M)

    return (value, a_values)
